# baseline (speedup 1.0000x reference)
import functools

import jax
import jax.numpy as jnp
from jax import lax
from jax.experimental import pallas as pl
from jax.experimental.pallas import tpu as pltpu
from jax.experimental.pallas import tpu_sc as plsc

N_NODES = 100000
D_STRUCT = 131
D_SPATIAL = 64
D_OUT = 256
D_PAD = 144

NC, NS = 2, 16
NW = NC * NS
N_PAD = 100096
B_PER_W = N_PAD // NW
CHUNK = 184
N_CHUNKS = B_PER_W // CHUNK

TC_BLOCK = 512


def _sc_gather_sum3(table, idx0, idx1, idx2):
    mesh = plsc.VectorSubcoreMesh(core_axis_name="c", subcore_axis_name="s")

    @functools.partial(
        pl.kernel,
        out_type=jax.ShapeDtypeStruct((N_PAD, D_PAD), jnp.float32),
        mesh=mesh,
        scratch_types=[
            pltpu.VMEM((CHUNK,), jnp.int32),
            pltpu.VMEM((CHUNK,), jnp.int32),
            pltpu.VMEM((CHUNK,), jnp.int32),
            pltpu.VMEM((CHUNK, D_PAD), jnp.float32),
            pltpu.VMEM((CHUNK, D_PAD), jnp.float32),
            pltpu.VMEM((CHUNK, D_PAD), jnp.float32),
            pltpu.VMEM((CHUNK, D_PAD), jnp.float32),
            pltpu.SemaphoreType.DMA,
        ],
        compiler_params=pltpu.CompilerParams(use_tc_tiling_on_sc=False),
    )
    def k(table_hbm, i0_hbm, i1_hbm, i2_hbm, out_hbm,
          i0_v, i1_v, i2_v, g0_v, g1_v, g2_v, acc_v, sem):
        wid = lax.axis_index("s") * NC + lax.axis_index("c")

        def chunk_body(c, carry):
            base = wid * B_PER_W + c * CHUNK
            pltpu.sync_copy(i0_hbm.at[pl.ds(base, CHUNK)], i0_v)
            pltpu.sync_copy(i1_hbm.at[pl.ds(base, CHUNK)], i1_v)
            pltpu.sync_copy(i2_hbm.at[pl.ds(base, CHUNK)], i2_v)
            d0 = pltpu.async_copy(table_hbm.at[i0_v], g0_v, sem)
            d1 = pltpu.async_copy(table_hbm.at[i1_v], g1_v, sem)
            d2 = pltpu.async_copy(table_hbm.at[i2_v], g2_v, sem)
            d0.wait()
            d1.wait()
            d2.wait()

            def row_body(j, c2):
                for d in range(D_PAD // 16):
                    sl = pl.ds(d * 16, 16)
                    acc_v[j, sl] = g0_v[j, sl] + g1_v[j, sl] + g2_v[j, sl]
                return c2

            lax.fori_loop(0, CHUNK, row_body, 0, unroll=2)
            pltpu.sync_copy(acc_v, out_hbm.at[pl.ds(base, CHUNK)])
            return carry

        lax.fori_loop(0, N_CHUNKS, chunk_body, 0)

    return k(table, idx0, idx1, idx2)


def _tc_body(sp_ref, st_ref, s3_ref, wcs_ref, wct_ref, bc_ref, wa_ref, ba_ref,
             o1_ref, o2_ref):
    sp = sp_ref[...]
    st = st_ref[...]
    s3 = s3_ref[...][:, :D_STRUCT]
    o1_ref[...] = (
        jnp.dot(sp, wcs_ref[...], preferred_element_type=jnp.float32)
        + jnp.dot(st, wct_ref[...], preferred_element_type=jnp.float32)
        + bc_ref[...]
    )
    x = (st + s3) * 0.25
    o2_ref[...] = (
        jnp.dot(x, wa_ref[...], preferred_element_type=jnp.float32) + ba_ref[...]
    )


def _tc_compute(spatial, structural, sum3, WcSp, WcSt, b_comb, WaT, b_agg):
    grid = (pl.cdiv(N_NODES, TC_BLOCK),)
    full = lambda i: (0, 0)
    return pl.pallas_call(
        _tc_body,
        grid=grid,
        in_specs=[
            pl.BlockSpec((TC_BLOCK, D_SPATIAL), lambda i: (i, 0)),
            pl.BlockSpec((TC_BLOCK, D_STRUCT), lambda i: (i, 0)),
            pl.BlockSpec((TC_BLOCK, D_PAD), lambda i: (i, 0)),
            pl.BlockSpec((D_SPATIAL, D_OUT), full),
            pl.BlockSpec((D_STRUCT, D_OUT), full),
            pl.BlockSpec((1, D_OUT), full),
            pl.BlockSpec((D_STRUCT, D_OUT), full),
            pl.BlockSpec((1, D_OUT), full),
        ],
        out_specs=[
            pl.BlockSpec((TC_BLOCK, D_OUT), lambda i: (i, 0)),
            pl.BlockSpec((TC_BLOCK, D_OUT), lambda i: (i, 0)),
        ],
        out_shape=[
            jax.ShapeDtypeStruct((N_NODES, D_OUT), jnp.float32),
            jax.ShapeDtypeStruct((N_NODES, D_OUT), jnp.float32),
        ],
    )(spatial, structural, sum3, WcSp, WcSt, b_comb, WaT, b_agg)


def kernel(spatial, structural, neighbour, W_comb, b_comb, W_agg, b_agg):
    idx = neighbour.astype(jnp.int32)
    pad = N_PAD - N_NODES
    idx0 = jnp.pad(idx[:, 0], (0, pad))
    idx1 = jnp.pad(idx[:, 1], (0, pad))
    idx2 = jnp.pad(idx[:, 2], (0, pad))
    table = jnp.pad(structural, ((0, 0), (0, D_PAD - D_STRUCT)))

    sum3 = _sc_gather_sum3(table, idx0, idx1, idx2)

    WcSp = W_comb[:, :D_SPATIAL].T
    WcSt = W_comb[:, D_SPATIAL:].T
    WaT = W_agg.T
    out1, out2 = _tc_compute(
        spatial, structural, sum3,
        WcSp, WcSt, b_comb.reshape(1, D_OUT), WaT, b_agg.reshape(1, D_OUT),
    )
    return (out1, out2)


# speedup vs baseline: 1.3483x; 1.3483x over previous
import functools

import jax
import jax.numpy as jnp
from jax import lax
from jax.experimental import pallas as pl
from jax.experimental.pallas import tpu as pltpu
from jax.experimental.pallas import tpu_sc as plsc

N_NODES = 100000
D_STRUCT = 131
D_SPATIAL = 64
D_OUT = 256

NC, NS = 2, 16
NW = NC * NS
B_PER_W = 3200
N_IDX = NW * B_PER_W
CHUNK = 80
SLICES = D_OUT // 16

TC_BLOCK = 512


def _sc_gather_mean(table, idx0, idx1, idx2):
    mesh = plsc.VectorSubcoreMesh(core_axis_name="c", subcore_axis_name="s")

    @functools.partial(
        pl.kernel,
        out_type=jax.ShapeDtypeStruct((N_NODES, D_OUT), jnp.float32),
        mesh=mesh,
        scratch_types=[
            pltpu.VMEM((B_PER_W,), jnp.int32),
            pltpu.VMEM((B_PER_W,), jnp.int32),
            pltpu.VMEM((B_PER_W,), jnp.int32),
            pltpu.VMEM((CHUNK, D_OUT), jnp.float32),
            pltpu.VMEM((CHUNK, D_OUT), jnp.float32),
            pltpu.VMEM((CHUNK, D_OUT), jnp.float32),
            pltpu.VMEM((CHUNK, D_OUT), jnp.float32),
            pltpu.SemaphoreType.DMA,
        ],
    )
    def k(table_hbm, i0_hbm, i1_hbm, i2_hbm, out_hbm,
          i0_v, i1_v, i2_v, g0_v, g1_v, g2_v, acc_v, sem):
        wid = lax.axis_index("s") * NC + lax.axis_index("c")
        wbase = wid * B_PER_W
        n_chunks = jnp.where(wid == NW - 1, 800 // CHUNK, B_PER_W // CHUNK)
        pltpu.sync_copy(i0_hbm.at[pl.ds(wbase, B_PER_W)], i0_v)
        pltpu.sync_copy(i1_hbm.at[pl.ds(wbase, B_PER_W)], i1_v)
        pltpu.sync_copy(i2_hbm.at[pl.ds(wbase, B_PER_W)], i2_v)

        def chunk_body(c, carry):
            lbase = c * CHUNK
            base = wbase + lbase
            dself = pltpu.async_copy(table_hbm.at[pl.ds(base, CHUNK)], acc_v, sem)
            d0 = pltpu.async_copy(
                table_hbm.at[i0_v.at[pl.ds(lbase, CHUNK)]], g0_v, sem)
            d1 = pltpu.async_copy(
                table_hbm.at[i1_v.at[pl.ds(lbase, CHUNK)]], g1_v, sem)
            d2 = pltpu.async_copy(
                table_hbm.at[i2_v.at[pl.ds(lbase, CHUNK)]], g2_v, sem)
            dself.wait()
            d0.wait()
            d1.wait()
            d2.wait()

            def row_body(j, c2):
                for d in range(SLICES):
                    sl = pl.ds(d * 16, 16)
                    acc_v[j, sl] = (
                        acc_v[j, sl] + g0_v[j, sl] + g1_v[j, sl] + g2_v[j, sl]
                    ) * 0.25
                return c2

            lax.fori_loop(0, CHUNK, row_body, 0, unroll=2)
            pltpu.sync_copy(acc_v, out_hbm.at[pl.ds(base, CHUNK)])
            return carry

        lax.fori_loop(0, n_chunks, chunk_body, 0)

    return k(table, idx0, idx1, idx2)


def _tc_body(sp_ref, st_ref, wcs_ref, wct_ref, bc_ref, wa_ref, ba_ref,
             o1_ref, p_ref):
    sp = sp_ref[...]
    st = st_ref[...]
    o1_ref[...] = (
        jnp.dot(sp, wcs_ref[...], preferred_element_type=jnp.float32)
        + jnp.dot(st, wct_ref[...], preferred_element_type=jnp.float32)
        + bc_ref[...]
    )
    p_ref[...] = (
        jnp.dot(st, wa_ref[...], preferred_element_type=jnp.float32) + ba_ref[...]
    )


def _tc_compute(spatial, structural, WcSp, WcSt, b_comb, WaT, b_agg):
    grid = (pl.cdiv(N_NODES, TC_BLOCK),)
    full = lambda i: (0, 0)
    return pl.pallas_call(
        _tc_body,
        grid=grid,
        in_specs=[
            pl.BlockSpec((TC_BLOCK, D_SPATIAL), lambda i: (i, 0)),
            pl.BlockSpec((TC_BLOCK, D_STRUCT), lambda i: (i, 0)),
            pl.BlockSpec((D_SPATIAL, D_OUT), full),
            pl.BlockSpec((D_STRUCT, D_OUT), full),
            pl.BlockSpec((1, D_OUT), full),
            pl.BlockSpec((D_STRUCT, D_OUT), full),
            pl.BlockSpec((1, D_OUT), full),
        ],
        out_specs=[
            pl.BlockSpec((TC_BLOCK, D_OUT), lambda i: (i, 0)),
            pl.BlockSpec((TC_BLOCK, D_OUT), lambda i: (i, 0)),
        ],
        out_shape=[
            jax.ShapeDtypeStruct((N_NODES, D_OUT), jnp.float32),
            jax.ShapeDtypeStruct((N_NODES, D_OUT), jnp.float32),
        ],
    )(spatial, structural, WcSp, WcSt, b_comb, WaT, b_agg)


def kernel(spatial, structural, neighbour, W_comb, b_comb, W_agg, b_agg):
    idx = neighbour.astype(jnp.int32)
    pad = N_IDX - N_NODES
    idx0 = jnp.pad(idx[:, 0], (0, pad))
    idx1 = jnp.pad(idx[:, 1], (0, pad))
    idx2 = jnp.pad(idx[:, 2], (0, pad))

    WcSp = W_comb[:, :D_SPATIAL].T
    WcSt = W_comb[:, D_SPATIAL:].T
    WaT = W_agg.T
    out1, P = _tc_compute(
        spatial, structural,
        WcSp, WcSt, b_comb.reshape(1, D_OUT), WaT, b_agg.reshape(1, D_OUT),
    )
    out2 = _sc_gather_mean(P, idx0, idx1, idx2)
    return (out1, out2)


# speedup vs baseline: 1.7119x; 1.2697x over previous
import functools

import jax
import jax.numpy as jnp
from jax import lax
from jax.experimental import pallas as pl
from jax.experimental.pallas import tpu as pltpu
from jax.experimental.pallas import tpu_sc as plsc

N_NODES = 100000
D_STRUCT = 131
D_SPATIAL = 64
D_OUT = 256

NC, NS = 2, 16
NW = NC * NS
B_PER_W = 3200
N_IDX = NW * B_PER_W
CHUNK = 80
SLICES = D_OUT // 16

TC_BLOCK = 512


def _sc_gather_mean(table, idx0, idx1, idx2):
    mesh = plsc.VectorSubcoreMesh(core_axis_name="c", subcore_axis_name="s")

    @functools.partial(
        pl.kernel,
        out_type=jax.ShapeDtypeStruct((N_NODES, D_OUT), jnp.float32),
        mesh=mesh,
        scratch_types=[
            pltpu.VMEM((B_PER_W,), jnp.int32),
            pltpu.VMEM((B_PER_W,), jnp.int32),
            pltpu.VMEM((B_PER_W,), jnp.int32),
            pltpu.VMEM((CHUNK, D_OUT), jnp.float32),
            pltpu.VMEM((CHUNK, D_OUT), jnp.float32),
            pltpu.VMEM((CHUNK, D_OUT), jnp.float32),
            pltpu.VMEM((CHUNK, D_OUT), jnp.float32),
            pltpu.SemaphoreType.DMA,
        ],
    )
    def k(table_hbm, i0_hbm, i1_hbm, i2_hbm, out_hbm,
          i0_v, i1_v, i2_v, g0_v, g1_v, g2_v, acc_v, sem):
        wid = lax.axis_index("s") * NC + lax.axis_index("c")
        wbase = wid * B_PER_W
        n_chunks = jnp.where(wid == NW - 1, 800 // CHUNK, B_PER_W // CHUNK)
        pltpu.sync_copy(i0_hbm.at[pl.ds(wbase, B_PER_W)], i0_v)
        pltpu.sync_copy(i1_hbm.at[pl.ds(wbase, B_PER_W)], i1_v)
        pltpu.sync_copy(i2_hbm.at[pl.ds(wbase, B_PER_W)], i2_v)

        def chunk_body(c, carry):
            lbase = c * CHUNK
            base = wbase + lbase
            dself = pltpu.async_copy(table_hbm.at[pl.ds(base, CHUNK)], acc_v, sem)
            d0 = pltpu.async_copy(
                table_hbm.at[i0_v.at[pl.ds(lbase, CHUNK)]], g0_v, sem)
            d1 = pltpu.async_copy(
                table_hbm.at[i1_v.at[pl.ds(lbase, CHUNK)]], g1_v, sem)
            d2 = pltpu.async_copy(
                table_hbm.at[i2_v.at[pl.ds(lbase, CHUNK)]], g2_v, sem)
            dself.wait()
            d0.wait()
            d1.wait()
            d2.wait()

            def row_body(j, c2):
                for d in range(SLICES):
                    sl = pl.ds(d * 16, 16)
                    acc_v[j, sl] = (
                        acc_v[j, sl] + g0_v[j, sl] + g1_v[j, sl] + g2_v[j, sl]
                    ) * 0.25
                return c2

            lax.fori_loop(0, CHUNK, row_body, 0, unroll=2)
            pltpu.sync_copy(acc_v, out_hbm.at[pl.ds(base, CHUNK)])
            return carry

        lax.fori_loop(0, n_chunks, chunk_body, 0)

    return k(table, idx0, idx1, idx2)


def _p_body(stt_ref, wa_ref, ba_ref, p_ref):
    p_ref[...] = (
        lax.dot_general(
            stt_ref[...], wa_ref[...],
            dimension_numbers=(((0,), (0,)), ((), ())),
            preferred_element_type=jnp.float32,
        )
        + ba_ref[...]
    )


def _tc_p(structural_t, WaT, b_agg):
    grid = (pl.cdiv(N_NODES, TC_BLOCK),)
    full = lambda i: (0, 0)
    return pl.pallas_call(
        _p_body,
        grid=grid,
        in_specs=[
            pl.BlockSpec((D_STRUCT, TC_BLOCK), lambda i: (0, i)),
            pl.BlockSpec((D_STRUCT, D_OUT), full),
            pl.BlockSpec((1, D_OUT), full),
        ],
        out_specs=pl.BlockSpec((TC_BLOCK, D_OUT), lambda i: (i, 0)),
        out_shape=jax.ShapeDtypeStruct((N_NODES, D_OUT), jnp.float32),
    )(structural_t, WaT, b_agg)


def _out1_body(spt_ref, stt_ref, wcs_ref, wct_ref, bc_ref, o1_ref):
    dn = (((0,), (0,)), ((), ()))
    o1_ref[...] = (
        lax.dot_general(spt_ref[...], wcs_ref[...], dimension_numbers=dn,
                        preferred_element_type=jnp.float32)
        + lax.dot_general(stt_ref[...], wct_ref[...], dimension_numbers=dn,
                          preferred_element_type=jnp.float32)
        + bc_ref[...]
    )


def _tc_out1(spatial_t, structural_t, WcSp, WcSt, b_comb):
    grid = (pl.cdiv(N_NODES, TC_BLOCK),)
    full = lambda i: (0, 0)
    return pl.pallas_call(
        _out1_body,
        grid=grid,
        in_specs=[
            pl.BlockSpec((D_SPATIAL, TC_BLOCK), lambda i: (0, i)),
            pl.BlockSpec((D_STRUCT, TC_BLOCK), lambda i: (0, i)),
            pl.BlockSpec((D_SPATIAL, D_OUT), full),
            pl.BlockSpec((D_STRUCT, D_OUT), full),
            pl.BlockSpec((1, D_OUT), full),
        ],
        out_specs=pl.BlockSpec((TC_BLOCK, D_OUT), lambda i: (i, 0)),
        out_shape=jax.ShapeDtypeStruct((N_NODES, D_OUT), jnp.float32),
    )(spatial_t, structural_t, WcSp, WcSt, b_comb)


def kernel(spatial, structural, neighbour, W_comb, b_comb, W_agg, b_agg):
    idx_t = neighbour.astype(jnp.int32).T
    pad = N_IDX - N_NODES
    idx0 = jnp.pad(idx_t[0], (0, pad))
    idx1 = jnp.pad(idx_t[1], (0, pad))
    idx2 = jnp.pad(idx_t[2], (0, pad))

    WcT = W_comb.T
    WcSp = WcT[:D_SPATIAL]
    WcSt = WcT[D_SPATIAL:]
    WaT = W_agg.T
    spatial_t = spatial.T
    structural_t = structural.T

    P = _tc_p(structural_t, WaT, b_agg.reshape(1, D_OUT))
    out2 = _sc_gather_mean(P, idx0, idx1, idx2)
    out1 = _tc_out1(spatial_t, structural_t, WcSp, WcSt,
                    b_comb.reshape(1, D_OUT))
    return (out1, out2)


# speedup vs baseline: 1.7248x; 1.0075x over previous
import functools

import jax
import jax.numpy as jnp
from jax import lax
from jax.experimental import pallas as pl
from jax.experimental.pallas import tpu as pltpu
from jax.experimental.pallas import tpu_sc as plsc

N_NODES = 100000
D_STRUCT = 131
D_SPATIAL = 64
D_OUT = 256

NC, NS = 2, 16
NW = NC * NS
B_PER_W = 3200
N_IDX = NW * B_PER_W
CHUNK = 40
SLICES = D_OUT // 16

TC_BLOCK = 512


def _sc_gather_mean(table, idx0, idx1, idx2):
    mesh = plsc.VectorSubcoreMesh(core_axis_name="c", subcore_axis_name="s")

    @functools.partial(
        pl.kernel,
        out_type=jax.ShapeDtypeStruct((N_NODES, D_OUT), jnp.float32),
        mesh=mesh,
        scratch_types=[
            pltpu.VMEM((B_PER_W,), jnp.int32),
            pltpu.VMEM((B_PER_W,), jnp.int32),
            pltpu.VMEM((B_PER_W,), jnp.int32),
            [pltpu.VMEM((CHUNK, D_OUT), jnp.float32) for _ in range(2)],
            [pltpu.VMEM((CHUNK, D_OUT), jnp.float32) for _ in range(2)],
            [pltpu.VMEM((CHUNK, D_OUT), jnp.float32) for _ in range(2)],
            [pltpu.VMEM((CHUNK, D_OUT), jnp.float32) for _ in range(2)],
            [pltpu.SemaphoreType.DMA for _ in range(2)],
        ],
    )
    def k(table_hbm, i0_hbm, i1_hbm, i2_hbm, out_hbm,
          i0_v, i1_v, i2_v, g0, g1, g2, acc, sems):
        wid = lax.axis_index("s") * NC + lax.axis_index("c")
        wbase = wid * B_PER_W
        n_chunks = jnp.where(wid == NW - 1, 800 // CHUNK, B_PER_W // CHUNK)
        pltpu.sync_copy(i0_hbm.at[pl.ds(wbase, B_PER_W)], i0_v)
        pltpu.sync_copy(i1_hbm.at[pl.ds(wbase, B_PER_W)], i1_v)
        pltpu.sync_copy(i2_hbm.at[pl.ds(wbase, B_PER_W)], i2_v)

        def issue4(c, b):
            lbase = c * CHUNK
            ds = pltpu.async_copy(
                table_hbm.at[pl.ds(wbase + lbase, CHUNK)], acc[b], sems[b])
            dg = [
                pltpu.async_copy(
                    table_hbm.at[iv.at[pl.ds(lbase, CHUNK)]], gk[b], sems[b])
                for gk, iv in zip((g0, g1, g2), (i0_v, i1_v, i2_v))
            ]
            return [ds] + dg

        def compute_and_store(c, b):
            def row_body(j, c2):
                for d in range(SLICES):
                    sl = pl.ds(d * 16, 16)
                    acc[b][j, sl] = (
                        acc[b][j, sl] + g0[b][j, sl] + g1[b][j, sl]
                        + g2[b][j, sl]
                    ) * 0.25
                return c2

            lax.fori_loop(0, CHUNK, row_body, 0, unroll=2)
            pltpu.sync_copy(acc[b], out_hbm.at[pl.ds(wbase + c * CHUNK, CHUNK)])

        def pair_body(i, carry):
            c0 = 2 * i
            da = issue4(c0, 0)
            db = issue4(c0 + 1, 1)
            for d in da:
                d.wait()
            compute_and_store(c0, 0)
            for d in db:
                d.wait()
            compute_and_store(c0 + 1, 1)
            return carry

        lax.fori_loop(0, n_chunks // 2, pair_body, 0)

    return k(table, idx0, idx1, idx2)


def _p_body(stt_ref, wa_ref, ba_ref, p_ref):
    p_ref[...] = (
        lax.dot_general(
            stt_ref[...], wa_ref[...],
            dimension_numbers=(((0,), (0,)), ((), ())),
            preferred_element_type=jnp.float32,
        )
        + ba_ref[...]
    )


def _tc_p(structural_t, WaT, b_agg):
    grid = (pl.cdiv(N_NODES, TC_BLOCK),)
    full = lambda i: (0, 0)
    return pl.pallas_call(
        _p_body,
        grid=grid,
        in_specs=[
            pl.BlockSpec((D_STRUCT, TC_BLOCK), lambda i: (0, i)),
            pl.BlockSpec((D_STRUCT, D_OUT), full),
            pl.BlockSpec((1, D_OUT), full),
        ],
        out_specs=pl.BlockSpec((TC_BLOCK, D_OUT), lambda i: (i, 0)),
        out_shape=jax.ShapeDtypeStruct((N_NODES, D_OUT), jnp.float32),
    )(structural_t, WaT, b_agg)


def _out1_body(spt_ref, stt_ref, wcs_ref, wct_ref, bc_ref, o1_ref):
    dn = (((0,), (0,)), ((), ()))
    o1_ref[...] = (
        lax.dot_general(spt_ref[...], wcs_ref[...], dimension_numbers=dn,
                        preferred_element_type=jnp.float32)
        + lax.dot_general(stt_ref[...], wct_ref[...], dimension_numbers=dn,
                          preferred_element_type=jnp.float32)
        + bc_ref[...]
    )


def _tc_out1(spatial_t, structural_t, WcSp, WcSt, b_comb):
    grid = (pl.cdiv(N_NODES, TC_BLOCK),)
    full = lambda i: (0, 0)
    return pl.pallas_call(
        _out1_body,
        grid=grid,
        in_specs=[
            pl.BlockSpec((D_SPATIAL, TC_BLOCK), lambda i: (0, i)),
            pl.BlockSpec((D_STRUCT, TC_BLOCK), lambda i: (0, i)),
            pl.BlockSpec((D_SPATIAL, D_OUT), full),
            pl.BlockSpec((D_STRUCT, D_OUT), full),
            pl.BlockSpec((1, D_OUT), full),
        ],
        out_specs=pl.BlockSpec((TC_BLOCK, D_OUT), lambda i: (i, 0)),
        out_shape=jax.ShapeDtypeStruct((N_NODES, D_OUT), jnp.float32),
    )(spatial_t, structural_t, WcSp, WcSt, b_comb)


def kernel(spatial, structural, neighbour, W_comb, b_comb, W_agg, b_agg):
    idx_t = neighbour.astype(jnp.int32).T
    pad = N_IDX - N_NODES
    idx0 = jnp.pad(idx_t[0], (0, pad))
    idx1 = jnp.pad(idx_t[1], (0, pad))
    idx2 = jnp.pad(idx_t[2], (0, pad))

    WcT = W_comb.T
    WcSp = WcT[:D_SPATIAL]
    WcSt = WcT[D_SPATIAL:]
    WaT = W_agg.T
    spatial_t = spatial.T
    structural_t = structural.T

    P = _tc_p(structural_t, WaT, b_agg.reshape(1, D_OUT))
    out2 = _sc_gather_mean(P, idx0, idx1, idx2)
    out1 = _tc_out1(spatial_t, structural_t, WcSp, WcSt,
                    b_comb.reshape(1, D_OUT))
    return (out1, out2)
